# Initial kernel scaffold; baseline (speedup 1.0000x reference)
import jax
import jax.numpy as jnp
from jax.experimental import pallas as pl


def kernel(hidden_states, gate_w, w_fc, w_proj):
    raise NotImplementedError("write your pallas kernel here")


# speedup vs baseline: 2.2883x; 2.2883x over previous
import functools

import jax
import jax.numpy as jnp
from jax import lax
from jax.experimental import pallas as pl
from jax.experimental.pallas import tpu as pltpu
from jax.experimental.pallas import tpu_sc as plsc

E = 8
TOP_K = 2
H = 1024
FF = 4096
NT = 4096
BM = 128
P = NT * TOP_K + E * BM
NB = P // BM

NC, NS = 2, 16
NW = NC * NS
TPW = NT // NW
CH = 64
CH2 = 32


def _router_body(x_ref, gt_ref, mx_ref, logits_ref, p0_ref, p1_ref,
                 g0_ref, g1_ref, bexp_ref):
    xb = x_ref[...].astype(jnp.bfloat16)
    gb = gt_ref[...].astype(jnp.bfloat16)
    l = lax.dot_general(xb, gb, (((1,), (0,)), ((), ())),
                        preferred_element_type=jnp.float32)
    logits_ref[...] = l
    m = jnp.max(l, axis=1, keepdims=True)
    ex = jnp.exp(l - m)
    probs = ex / jnp.sum(ex, axis=1, keepdims=True)
    eids = lax.broadcasted_iota(jnp.int32, (NT, E), 1)
    m1 = jnp.max(l, axis=1, keepdims=True)
    i1 = jnp.min(jnp.where(l == m1, eids, E), axis=1, keepdims=True)
    l2 = jnp.where(eids == i1, -jnp.inf, l)
    m2 = jnp.max(l2, axis=1, keepdims=True)
    i2 = jnp.min(jnp.where(l2 == m2, eids, E), axis=1, keepdims=True)
    g0_ref[...] = jnp.sum(jnp.where(eids == i1, probs, 0.0), axis=1,
                          keepdims=True)
    g1_ref[...] = jnp.sum(jnp.where(eids == i2, probs, 0.0), axis=1,
                          keepdims=True)
    oh0 = (eids == i1).astype(jnp.int32)
    oh1 = (eids == i2).astype(jnp.int32)
    both = oh0 + oh1
    inc = both
    tids = lax.broadcasted_iota(jnp.int32, (NT, E), 0)
    sh = 1
    while sh < NT:
        rolled = pltpu.roll(inc, sh, axis=0)
        inc = inc + jnp.where(tids >= sh, rolled, 0)
        sh *= 2
    excl = inc - both
    totals = lax.slice(inc, (NT - 1, 0), (NT, E))
    padded = ((totals + (BM - 1)) // BM) * BM
    offs = lax.dot_general(padded.astype(jnp.float32), mx_ref[...],
                           (((1,), (0,)), ((), ())),
                           preferred_element_type=jnp.float32,
                           precision=lax.Precision.HIGHEST).astype(jnp.int32)
    base = offs + excl
    p0_ref[...] = jnp.sum(oh0 * base, axis=1, keepdims=True)
    p1_ref[...] = jnp.sum(oh1 * base, axis=1, keepdims=True)
    used_cum = offs + padded
    brow = lax.broadcasted_iota(jnp.int32, (NB, E), 0) * BM
    bexp = jnp.sum((brow >= used_cum).astype(jnp.int32), axis=1,
                   keepdims=True)
    bexp_ref[...] = jnp.minimum(bexp, E - 1)


def _router(x, gate_t, mexcl):
    return pl.pallas_call(
        _router_body,
        out_shape=(
            jax.ShapeDtypeStruct((NT, E), jnp.float32),
            jax.ShapeDtypeStruct((NT, 1), jnp.int32),
            jax.ShapeDtypeStruct((NT, 1), jnp.int32),
            jax.ShapeDtypeStruct((NT, 1), jnp.float32),
            jax.ShapeDtypeStruct((NT, 1), jnp.float32),
            jax.ShapeDtypeStruct((NB, 1), jnp.int32),
        ),
        compiler_params=pltpu.CompilerParams(
            vmem_limit_bytes=100 * 1024 * 1024),
    )(x, gate_t, mexcl)


def _dispatch_body(x_hbm, p0_hbm, p1_hbm, xs_hbm, xbuf, i0, i1, sem):
    wid = lax.axis_index("s") * NC + lax.axis_index("c")
    base = wid * TPW
    for k in range(TPW // CH):
        tok = base + k * CH
        pltpu.sync_copy(x_hbm.at[pl.ds(tok, CH)], xbuf)
        pltpu.sync_copy(p0_hbm.at[pl.ds(tok, CH)], i0)
        pltpu.sync_copy(p1_hbm.at[pl.ds(tok, CH)], i1)
        d0 = pltpu.async_copy(xbuf, xs_hbm.at[i0], sem)
        d1 = pltpu.async_copy(xbuf, xs_hbm.at[i1], sem)
        d0.wait()
        d1.wait()


def _dispatch(x, p0, p1):
    mesh = plsc.VectorSubcoreMesh(core_axis_name="c", subcore_axis_name="s")
    f = pl.kernel(
        _dispatch_body,
        out_type=jax.ShapeDtypeStruct((P, H), jnp.float32),
        mesh=mesh,
        scratch_types=[
            pltpu.VMEM((CH, H), jnp.float32),
            pltpu.VMEM((CH,), jnp.int32),
            pltpu.VMEM((CH,), jnp.int32),
            pltpu.SemaphoreType.DMA,
        ],
    )
    return f(x, p0, p1)


def _gelu(h):
    return 0.5 * h * (1.0 + lax.erf(h * 0.7071067811865476))


NFF = 2
FC = FF // NFF


def _ffn_body(bexp_ref, xs_ref, wfc_ref, wpj_ref, o_ref):
    del bexp_ref
    k = pl.program_id(1)
    xb = xs_ref[...]
    h = lax.dot_general(xb, wfc_ref[0], (((1,), (1,)), ((), ())),
                        preferred_element_type=jnp.float32)
    h = _gelu(h)
    o = lax.dot_general(h, wpj_ref[0], (((1,), (1,)), ((), ())),
                        preferred_element_type=jnp.float32)

    @pl.when(k == 0)
    def _init():
        o_ref[...] = o

    @pl.when(k != 0)
    def _acc():
        o_ref[...] += o


def _ffn(bexp, xs, wfc, wpj):
    grid_spec = pltpu.PrefetchScalarGridSpec(
        num_scalar_prefetch=1,
        grid=(NB, NFF),
        in_specs=[
            pl.BlockSpec((BM, H), lambda b, k, be: (b, 0)),
            pl.BlockSpec((1, FC, H), lambda b, k, be: (be[b], k, 0)),
            pl.BlockSpec((1, H, FC), lambda b, k, be: (be[b], 0, k)),
        ],
        out_specs=pl.BlockSpec((BM, H), lambda b, k, be: (b, 0)),
    )
    return pl.pallas_call(
        _ffn_body,
        grid_spec=grid_spec,
        out_shape=jax.ShapeDtypeStruct((P, H), jnp.float32),
        compiler_params=pltpu.CompilerParams(
            vmem_limit_bytes=100 * 1024 * 1024),
    )(bexp, xs, wfc, wpj)


def _combine_body(o_hbm, p0_hbm, p1_hbm, oa_hbm, ob_hbm,
                  abuf, bbuf, i0, i1, sem):
    wid = lax.axis_index("s") * NC + lax.axis_index("c")
    base = wid * TPW
    for k in range(TPW // CH2):
        tok = base + k * CH2
        pltpu.sync_copy(p0_hbm.at[pl.ds(tok, CH2)], i0)
        pltpu.sync_copy(p1_hbm.at[pl.ds(tok, CH2)], i1)
        da = pltpu.async_copy(o_hbm.at[i0], abuf, sem)
        db = pltpu.async_copy(o_hbm.at[i1], bbuf, sem)
        da.wait()
        db.wait()
        pltpu.sync_copy(abuf, oa_hbm.at[pl.ds(tok, CH2)])
        pltpu.sync_copy(bbuf, ob_hbm.at[pl.ds(tok, CH2)])


def _combine(o, p0, p1):
    mesh = plsc.VectorSubcoreMesh(core_axis_name="c", subcore_axis_name="s")
    f = pl.kernel(
        _combine_body,
        out_type=(
            jax.ShapeDtypeStruct((NT, H), jnp.float32),
            jax.ShapeDtypeStruct((NT, H), jnp.float32),
        ),
        mesh=mesh,
        scratch_types=[
            pltpu.VMEM((CH2, H), jnp.float32),
            pltpu.VMEM((CH2, H), jnp.float32),
            pltpu.VMEM((CH2,), jnp.int32),
            pltpu.VMEM((CH2,), jnp.int32),
            pltpu.SemaphoreType.DMA,
        ],
    )
    return f(o, p0, p1)


def _add_body(a_ref, b_ref, ga_ref, gb_ref, o_ref):
    o_ref[...] = a_ref[...] * ga_ref[...] + b_ref[...] * gb_ref[...]


def _add(a, b, ga, gb):
    return pl.pallas_call(
        _add_body,
        grid=(NT // 256,),
        in_specs=[pl.BlockSpec((256, H), lambda i: (i, 0)),
                  pl.BlockSpec((256, H), lambda i: (i, 0)),
                  pl.BlockSpec((256, 1), lambda i: (i, 0)),
                  pl.BlockSpec((256, 1), lambda i: (i, 0))],
        out_specs=pl.BlockSpec((256, H), lambda i: (i, 0)),
        out_shape=jax.ShapeDtypeStruct((NT, H), jnp.float32),
    )(a, b, ga, gb)


def kernel(hidden_states, gate_w, w_fc, w_proj):
    Bs, Ss, Hs = hidden_states.shape
    x = hidden_states.reshape(-1, Hs)
    gate_t = gate_w.T
    r = jnp.arange(E)
    mexcl = (r[:, None] < r[None, :]).astype(jnp.float32)
    logits, pos0, pos1, g0, g1, bexp = _router(x, gate_t, mexcl)
    p0 = pos0.reshape(NT)
    p1 = pos1.reshape(NT)
    xs = _dispatch(x, p0, p1)
    o = _ffn(bexp.reshape(NB), xs, w_fc, w_proj)
    oa, ob = _combine(o, p0, p1)
    out = _add(oa, ob, g0, g1).reshape(Bs, Ss, Hs)
    return out, logits


# speedup vs baseline: 2.8544x; 1.2474x over previous
import functools

import jax
import jax.numpy as jnp
from jax import lax
from jax.experimental import pallas as pl
from jax.experimental.pallas import tpu as pltpu
from jax.experimental.pallas import tpu_sc as plsc

E = 8
TOP_K = 2
H = 1024
FF = 4096
NT = 4096
BM = 128
P = NT * TOP_K + E * BM
NB = P // BM

NC, NS = 2, 16
NW = NC * NS
TPW = NT // NW
CH = 64
CH2 = 32


def _router_body(x_ref, gt_ref, mx_ref, logits_ref, p0_ref, p1_ref,
                 g0_ref, g1_ref, bexp_ref):
    xb = x_ref[...].astype(jnp.bfloat16)
    gb = gt_ref[...].astype(jnp.bfloat16)
    l = lax.dot_general(xb, gb, (((1,), (0,)), ((), ())),
                        preferred_element_type=jnp.float32)
    logits_ref[...] = l
    m = jnp.max(l, axis=1, keepdims=True)
    ex = jnp.exp(l - m)
    probs = ex / jnp.sum(ex, axis=1, keepdims=True)
    eids = lax.broadcasted_iota(jnp.int32, (NT, E), 1)
    m1 = jnp.max(l, axis=1, keepdims=True)
    i1 = jnp.min(jnp.where(l == m1, eids, E), axis=1, keepdims=True)
    l2 = jnp.where(eids == i1, -jnp.inf, l)
    m2 = jnp.max(l2, axis=1, keepdims=True)
    i2 = jnp.min(jnp.where(l2 == m2, eids, E), axis=1, keepdims=True)
    g0_ref[...] = jnp.sum(jnp.where(eids == i1, probs, 0.0), axis=1,
                          keepdims=True)
    g1_ref[...] = jnp.sum(jnp.where(eids == i2, probs, 0.0), axis=1,
                          keepdims=True)
    oh0 = (eids == i1).astype(jnp.int32)
    oh1 = (eids == i2).astype(jnp.int32)
    both = oh0 + oh1
    inc = both
    tids = lax.broadcasted_iota(jnp.int32, (NT, E), 0)
    sh = 1
    while sh < NT:
        rolled = pltpu.roll(inc, sh, axis=0)
        inc = inc + jnp.where(tids >= sh, rolled, 0)
        sh *= 2
    excl = inc - both
    totals = lax.slice(inc, (NT - 1, 0), (NT, E))
    padded = ((totals + (BM - 1)) // BM) * BM
    offs = lax.dot_general(padded.astype(jnp.float32), mx_ref[...],
                           (((1,), (0,)), ((), ())),
                           preferred_element_type=jnp.float32,
                           precision=lax.Precision.HIGHEST).astype(jnp.int32)
    base = offs + excl
    p0_ref[...] = jnp.sum(oh0 * base, axis=1, keepdims=True)
    p1_ref[...] = jnp.sum(oh1 * base, axis=1, keepdims=True)
    used_cum = offs + padded
    brow = lax.broadcasted_iota(jnp.int32, (NB, E), 0) * BM
    bexp = jnp.sum((brow >= used_cum).astype(jnp.int32), axis=1,
                   keepdims=True)
    bexp_ref[...] = jnp.minimum(bexp, E - 1)


def _router(x, gate_t, mexcl):
    return pl.pallas_call(
        _router_body,
        out_shape=(
            jax.ShapeDtypeStruct((NT, E), jnp.float32),
            jax.ShapeDtypeStruct((NT, 1), jnp.int32),
            jax.ShapeDtypeStruct((NT, 1), jnp.int32),
            jax.ShapeDtypeStruct((NT, 1), jnp.float32),
            jax.ShapeDtypeStruct((NT, 1), jnp.float32),
            jax.ShapeDtypeStruct((NB, 1), jnp.int32),
        ),
        compiler_params=pltpu.CompilerParams(
            vmem_limit_bytes=100 * 1024 * 1024),
    )(x, gate_t, mexcl)


def _dispatch_body(x_hbm, p0_hbm, p1_hbm, xs_hbm, xbuf, i0, i1, sem):
    wid = lax.axis_index("s") * NC + lax.axis_index("c")
    base = wid * TPW
    for k in range(TPW // CH):
        tok = base + k * CH
        pltpu.sync_copy(x_hbm.at[pl.ds(tok, CH)], xbuf)
        pltpu.sync_copy(p0_hbm.at[pl.ds(tok, CH)], i0)
        pltpu.sync_copy(p1_hbm.at[pl.ds(tok, CH)], i1)
        d0 = pltpu.async_copy(xbuf, xs_hbm.at[i0], sem)
        d1 = pltpu.async_copy(xbuf, xs_hbm.at[i1], sem)
        d0.wait()
        d1.wait()


def _dispatch(x, p0, p1):
    mesh = plsc.VectorSubcoreMesh(core_axis_name="c", subcore_axis_name="s")
    f = pl.kernel(
        _dispatch_body,
        out_type=jax.ShapeDtypeStruct((P, H), jnp.float32),
        mesh=mesh,
        scratch_types=[
            pltpu.VMEM((CH, H), jnp.float32),
            pltpu.VMEM((CH,), jnp.int32),
            pltpu.VMEM((CH,), jnp.int32),
            pltpu.SemaphoreType.DMA,
        ],
    )
    return f(x, p0, p1)


def _gelu(h):
    return 0.5 * h * (1.0 + lax.erf(h * 0.7071067811865476))


def _ffn_body(bexp_ref, xs_ref, wfc_ref, wpj_ref, o_ref):
    del bexp_ref
    xb = xs_ref[...].astype(jnp.bfloat16)
    h = lax.dot_general(xb, wfc_ref[0], (((1,), (1,)), ((), ())),
                        preferred_element_type=jnp.float32)
    h = _gelu(h).astype(jnp.bfloat16)
    o = lax.dot_general(h, wpj_ref[0], (((1,), (1,)), ((), ())),
                        preferred_element_type=jnp.float32)
    o_ref[...] = o


def _ffn(bexp, xs, wfc, wpj):
    grid_spec = pltpu.PrefetchScalarGridSpec(
        num_scalar_prefetch=1,
        grid=(NB,),
        in_specs=[
            pl.BlockSpec((BM, H), lambda b, be: (b, 0)),
            pl.BlockSpec((1, FF, H), lambda b, be: (be[b], 0, 0)),
            pl.BlockSpec((1, H, FF), lambda b, be: (be[b], 0, 0)),
        ],
        out_specs=pl.BlockSpec((BM, H), lambda b, be: (b, 0)),
    )
    return pl.pallas_call(
        _ffn_body,
        grid_spec=grid_spec,
        out_shape=jax.ShapeDtypeStruct((P, H), jnp.float32),
        compiler_params=pltpu.CompilerParams(
            vmem_limit_bytes=100 * 1024 * 1024),
    )(bexp, xs, wfc, wpj)


def _combine_body(o_hbm, p0_hbm, p1_hbm, oa_hbm, ob_hbm,
                  abuf, bbuf, i0, i1, sem):
    wid = lax.axis_index("s") * NC + lax.axis_index("c")
    base = wid * TPW
    for k in range(TPW // CH2):
        tok = base + k * CH2
        pltpu.sync_copy(p0_hbm.at[pl.ds(tok, CH2)], i0)
        pltpu.sync_copy(p1_hbm.at[pl.ds(tok, CH2)], i1)
        da = pltpu.async_copy(o_hbm.at[i0], abuf, sem)
        db = pltpu.async_copy(o_hbm.at[i1], bbuf, sem)
        da.wait()
        db.wait()
        pltpu.sync_copy(abuf, oa_hbm.at[pl.ds(tok, CH2)])
        pltpu.sync_copy(bbuf, ob_hbm.at[pl.ds(tok, CH2)])


def _combine(o, p0, p1):
    mesh = plsc.VectorSubcoreMesh(core_axis_name="c", subcore_axis_name="s")
    f = pl.kernel(
        _combine_body,
        out_type=(
            jax.ShapeDtypeStruct((NT, H), jnp.float32),
            jax.ShapeDtypeStruct((NT, H), jnp.float32),
        ),
        mesh=mesh,
        scratch_types=[
            pltpu.VMEM((CH2, H), jnp.float32),
            pltpu.VMEM((CH2, H), jnp.float32),
            pltpu.VMEM((CH2,), jnp.int32),
            pltpu.VMEM((CH2,), jnp.int32),
            pltpu.SemaphoreType.DMA,
        ],
    )
    return f(o, p0, p1)


def _add_body(a_ref, b_ref, ga_ref, gb_ref, o_ref):
    o_ref[...] = a_ref[...] * ga_ref[...] + b_ref[...] * gb_ref[...]


def _add(a, b, ga, gb):
    return pl.pallas_call(
        _add_body,
        grid=(NT // 256,),
        in_specs=[pl.BlockSpec((256, H), lambda i: (i, 0)),
                  pl.BlockSpec((256, H), lambda i: (i, 0)),
                  pl.BlockSpec((256, 1), lambda i: (i, 0)),
                  pl.BlockSpec((256, 1), lambda i: (i, 0))],
        out_specs=pl.BlockSpec((256, H), lambda i: (i, 0)),
        out_shape=jax.ShapeDtypeStruct((NT, H), jnp.float32),
    )(a, b, ga, gb)


def kernel(hidden_states, gate_w, w_fc, w_proj):
    Bs, Ss, Hs = hidden_states.shape
    x = hidden_states.reshape(-1, Hs)
    gate_t = gate_w.T
    r = jnp.arange(E)
    mexcl = (r[:, None] < r[None, :]).astype(jnp.float32)
    logits, pos0, pos1, g0, g1, bexp = _router(x, gate_t, mexcl)
    p0 = pos0.reshape(NT)
    p1 = pos1.reshape(NT)
    xs = _dispatch(x, p0, p1)
    o = _ffn(bexp.reshape(NB), xs, w_fc.astype(jnp.bfloat16),
             w_proj.astype(jnp.bfloat16))
    oa, ob = _combine(o, p0, p1)
    out = _add(oa, ob, g0, g1).reshape(Bs, Ss, Hs)
    return out, logits
